# baseline (speedup 1.0000x reference)
import functools

import jax
import jax.numpy as jnp
import numpy as np
from jax import lax
from jax.experimental import pallas as pl
from jax.experimental.pallas import tpu as pltpu
from jax.experimental.pallas import tpu_sc as plsc


def _sc_gather_kernel(labels_hbm, starts_hbm, w_hbm, out_hbm,
                      starts_v, labv, rows_v, sem):
    wid = (lax.axis_index("c") == 0) & (lax.axis_index("s") == 0)

    @pl.when(wid)
    def _():
        pltpu.sync_copy(starts_hbm, starts_v)
        pltpu.async_copy(labels_hbm.at[starts_v], labv, sem).wait()
        pltpu.async_copy(w_hbm.at[labv], rows_v, sem).wait()
        pltpu.sync_copy(rows_v, out_hbm)


def _gather_bag_embeddings(labels, starts, W):
    B = starts.shape[0]
    C, H = W.shape
    mesh = plsc.VectorSubcoreMesh(core_axis_name="c", subcore_axis_name="s")
    k = functools.partial(
        pl.kernel,
        mesh=mesh,
        out_type=jax.ShapeDtypeStruct((B, H), jnp.float32),
        scratch_types=[
            pltpu.VMEM((B,), jnp.int32),
            pltpu.VMEM((B,), jnp.int32),
            pltpu.VMEM((B, H), jnp.float32),
            pltpu.SemaphoreType.DMA,
        ],
    )(_sc_gather_kernel)
    return k(labels, starts, W)


def _att_bag_kernel(x_ref, wbag_ref, w_ref, b_ref, repre_ref, logits_ref):
    H = w_ref.shape[1]
    C = w_ref.shape[0]
    w = wbag_ref[...].reshape(1, H)
    x = x_ref[...]
    logit = jax.lax.dot_general(
        x, w, (((1,), (1,)), ((), ())), preferred_element_type=jnp.float32
    )
    m = jnp.max(logit)
    p = jnp.exp(logit - m)
    s = jnp.sum(p)
    acc = jax.lax.dot_general(
        p, x, (((0,), (0,)), ((), ())), preferred_element_type=jnp.float32
    )
    repre = acc * (1.0 / s)
    repre_ref[...] = repre.reshape(1, 1, H)
    row = jax.lax.dot_general(
        repre, w_ref[...], (((1,), (1,)), ((), ())),
        preferred_element_type=jnp.float32,
    ) + b_ref[...]
    logits_ref[...] = row.reshape(1, 1, C)


def kernel(x, labels, scopes, W, b):
    N, H = x.shape
    C = W.shape[0]
    B = scopes.shape[0]
    L = N // B

    starts = jnp.asarray(scopes)[:, 0].astype(jnp.int32)
    w_bag = _gather_bag_embeddings(labels.astype(jnp.int32), starts, W)
    b2 = b.reshape(1, C)

    repre3, logits3 = pl.pallas_call(
        _att_bag_kernel,
        grid=(B,),
        in_specs=[
            pl.BlockSpec((L, H), lambda i: (i, 0)),
            pl.BlockSpec((1, 1, H), lambda i: (i, 0, 0)),
            pl.BlockSpec((C, H), lambda i: (0, 0)),
            pl.BlockSpec((1, C), lambda i: (0, 0)),
        ],
        out_specs=[
            pl.BlockSpec((1, 1, H), lambda i: (i, 0, 0)),
            pl.BlockSpec((1, 1, C), lambda i: (i, 0, 0)),
        ],
        out_shape=[
            jax.ShapeDtypeStruct((B, 1, H), jnp.float32),
            jax.ShapeDtypeStruct((B, 1, C), jnp.float32),
        ],
        compiler_params=pltpu.CompilerParams(
            dimension_semantics=("parallel",)
        ),
    )(x, w_bag.reshape(B, 1, H), W, b2)
    return (repre3.reshape(B, H), logits3.reshape(B, C))


# speedup vs baseline: 1.4489x; 1.4489x over previous
import jax
import jax.numpy as jnp
import numpy as np
from jax.experimental import pallas as pl
from jax.experimental.pallas import tpu as pltpu


def _att_bag_kernel(bag_labels_ref, xa_ref, xb_ref, w_ref, b_ref,
                    repre_ref, logits_ref):
    i = pl.program_id(0)
    C = w_ref.shape[0]
    H = w_ref.shape[1]

    def one_bag(lab, x):
        onehot = (jax.lax.broadcasted_iota(jnp.int32, (1, C), 1) == lab
                  ).astype(jnp.float32)
        w = jax.lax.dot_general(
            onehot, w_ref[...], (((1,), (0,)), ((), ())),
            preferred_element_type=jnp.float32,
        )
        logit = jax.lax.dot_general(
            x, w, (((1,), (1,)), ((), ())), preferred_element_type=jnp.float32
        )
        m = jnp.max(logit)
        p = jnp.exp(logit - m)
        s = jnp.sum(p)
        acc = jax.lax.dot_general(
            p, x, (((0,), (0,)), ((), ())), preferred_element_type=jnp.float32
        )
        repre = acc * (1.0 / s)
        row = jax.lax.dot_general(
            repre, w_ref[...], (((1,), (1,)), ((), ())),
            preferred_element_type=jnp.float32,
        ) + b_ref[...]
        return repre, row

    ra, rowa = one_bag(bag_labels_ref[2 * i], xa_ref[...])
    rb, rowb = one_bag(bag_labels_ref[2 * i + 1], xb_ref[...])
    repre_ref[...] = jnp.concatenate([ra, rb], axis=0).reshape(2, 1, H)
    logits_ref[...] = jnp.concatenate([rowa, rowb], axis=0).reshape(2, 1, C)


def kernel(x, labels, scopes, W, b):
    N, H = x.shape
    C = W.shape[0]
    B = scopes.shape[0]
    L = N // B

    starts = jnp.asarray(scopes)[:, 0].astype(jnp.int32)
    bag_labels = jnp.take(labels, starts, axis=0).astype(jnp.int32)
    b2 = b.reshape(1, C)

    grid_spec = pltpu.PrefetchScalarGridSpec(
        num_scalar_prefetch=1,
        grid=(B // 2,),
        in_specs=[
            pl.BlockSpec((L, H), lambda i, *_: (2 * i, 0)),
            pl.BlockSpec((L, H), lambda i, *_: (2 * i + 1, 0)),
            pl.BlockSpec((C, H), lambda i, *_: (0, 0)),
            pl.BlockSpec((1, C), lambda i, *_: (0, 0)),
        ],
        out_specs=[
            pl.BlockSpec((2, 1, H), lambda i, *_: (i, 0, 0)),
            pl.BlockSpec((2, 1, C), lambda i, *_: (i, 0, 0)),
        ],
    )
    repre3, logits3 = pl.pallas_call(
        _att_bag_kernel,
        grid_spec=grid_spec,
        out_shape=[
            jax.ShapeDtypeStruct((B, 1, H), jnp.float32),
            jax.ShapeDtypeStruct((B, 1, C), jnp.float32),
        ],
        compiler_params=pltpu.CompilerParams(
            dimension_semantics=("parallel",)
        ),
    )(bag_labels, x, x, W, b2)
    return (repre3.reshape(B, H), logits3.reshape(B, C))
